# baseline (speedup 1.0000x reference)
import functools
import jax
import jax.numpy as jnp
import numpy as np
from jax import lax
from jax.experimental import pallas as pl
from jax.experimental.pallas import tpu as pltpu
from jax.experimental.pallas import tpu_sc as plsc

N_LOW = 10000
N_HIGH = 50000

NC, NS, LN = 2, 16, 16
NW = NC * NS
NEG_INF = float("-inf")


def _rup(x, m):
    return (x + m - 1) // m * m


def _mesh():
    return plsc.VectorSubcoreMesh(core_axis_name="c", subcore_axis_name="s")


_SC_PARAMS = pltpu.CompilerParams(use_tc_tiling_on_sc=False,
                                  needs_layout_passes=False)


def _wid():
    return lax.axis_index("s") * NC + lax.axis_index("c")


def _iota16():
    return jnp.arange(LN, dtype=jnp.int32)


def _take16(scr, x, idx):
    scr[...] = x
    return plsc.load_gather(scr, [idx])


def _shift_up(scr, x, s, ident):
    io = _iota16()
    g = _take16(scr, x, jnp.maximum(io - s, 0))
    return jnp.where(io >= s, g, ident)


def _last_mask(scr_i, d):
    io = _iota16()
    nxt = _take16(scr_i, d, jnp.minimum(io + 1, LN - 1))
    return (io == LN - 1) | (d != nxt)


def _mm_body(x_ref, w_ref, o_ref):
    o_ref[...] = jnp.dot(x_ref[...], w_ref[...],
                         preferred_element_type=jnp.float32)


def tc_matmul(x, w, bm=1024):
    m, k = x.shape
    n = w.shape[1]
    mp = _rup(m, bm)
    if mp != m:
        x = jnp.pad(x, ((0, mp - m), (0, 0)))
    out = pl.pallas_call(
        _mm_body,
        grid=(mp // bm,),
        in_specs=[pl.BlockSpec((bm, k), lambda i: (i, 0)),
                  pl.BlockSpec((k, n), lambda i: (0, 0))],
        out_specs=pl.BlockSpec((bm, n), lambda i: (i, 0)),
        out_shape=jax.ShapeDtypeStruct((mp, n), jnp.float32),
    )(x, w)
    return out[:m]


def _alpha_body(el_ref, er_ref, a_ref, o_ref):
    x = el_ref[...] + er_ref[...]
    g = jnp.maximum(x, 0.2 * x)
    o_ref[...] = jnp.dot(g, a_ref[...], preferred_element_type=jnp.float32)


def tc_alpha(el, er, amat, be=2048):
    ep, fp = el.shape
    h = amat.shape[1]
    return pl.pallas_call(
        _alpha_body,
        grid=(ep // be,),
        in_specs=[pl.BlockSpec((be, fp), lambda i: (i, 0)),
                  pl.BlockSpec((be, fp), lambda i: (i, 0)),
                  pl.BlockSpec((fp, h), lambda i: (0, 0))],
        out_specs=pl.BlockSpec((be, h), lambda i: (i, 0)),
        out_shape=jax.ShapeDtypeStruct((ep, h), jnp.float32),
    )(el, er, amat)


def tc_scale(el, al, heads, be=2048):
    ep, fp = el.shape
    cc = fp // heads

    def body(el_ref, al_ref, o_ref):
        parts = [el_ref[:, k * cc:(k + 1) * cc] * al_ref[:, k:k + 1]
                 for k in range(heads)]
        o_ref[...] = jnp.concatenate(parts, axis=1) if heads > 1 else parts[0]

    return pl.pallas_call(
        body,
        grid=(ep // be,),
        in_specs=[pl.BlockSpec((be, fp), lambda i: (i, 0)),
                  pl.BlockSpec((be, heads), lambda i: (i, 0))],
        out_specs=pl.BlockSpec((be, fp), lambda i: (i, 0)),
        out_shape=jax.ShapeDtypeStruct((ep, fp), jnp.float32),
    )(el, al)


def tc_merge(partials, op, bm=1024):
    nw, m = partials.shape

    def body(p_ref, o_ref):
        x = p_ref[...]
        if op == "max":
            r = jnp.max(x, axis=0)
            r = jnp.where(jnp.isfinite(r), r, 0.0)
        else:
            r = jnp.sum(x, axis=0)
        o_ref[...] = r[None, :]

    out = pl.pallas_call(
        body,
        grid=(m // bm,),
        in_specs=[pl.BlockSpec((nw, bm), lambda i: (0, i))],
        out_specs=pl.BlockSpec((1, bm), lambda i: (0, i)),
        out_shape=jax.ShapeDtypeStruct((1, m), jnp.float32),
    )(partials)
    return out[0]


def sc_gather2(tabl, tabr, idxl, idxr, cb=128, kf=2):
    ep = idxl.shape[0]
    fp = tabl.shape[1]
    per_w = ep // NW
    cg = cb * kf
    n_ch = per_w // cg

    def body(tl, tr, il, ir, ol, outr, ixl, ixr, rowsl, rowsr, sem):
        base = _wid() * per_w

        def chunk(i, _):
            off = base + i * cg
            for j in range(kf):
                pltpu.sync_copy(il.at[pl.ds(off + j * cb, cb)], ixl.at[j])
                pltpu.sync_copy(ir.at[pl.ds(off + j * cb, cb)], ixr.at[j])
            descs = []
            for j in range(kf):
                descs.append(pltpu.async_copy(
                    tl.at[ixl.at[j]], rowsl.at[pl.ds(j * cb, cb)], sem))
                descs.append(pltpu.async_copy(
                    tr.at[ixr.at[j]], rowsr.at[pl.ds(j * cb, cb)], sem))
            for d in descs:
                d.wait()
            pltpu.sync_copy(rowsl, ol.at[pl.ds(off, cg)])
            pltpu.sync_copy(rowsr, outr.at[pl.ds(off, cg)])
            return 0

        lax.fori_loop(0, n_ch, chunk, 0)

    shp = jax.ShapeDtypeStruct((ep, fp), jnp.float32)
    return pl.kernel(
        body,
        out_type=(shp, shp),
        mesh=_mesh(),
        name="scg%d" % fp,
        compiler_params=_SC_PARAMS,
        scratch_types=[pltpu.VMEM((kf, cb), jnp.int32),
                       pltpu.VMEM((kf, cb), jnp.int32),
                       pltpu.VMEM((cg, fp), jnp.float32),
                       pltpu.VMEM((cg, fp), jnp.float32),
                       pltpu.SemaphoreType.DMA],
    )(tabl, tabr, idxl, idxr)


def sc_stats(dstp, vals, nt, h, op, use_ones=False, cb=512):
    ep = dstp.shape[0]
    per_w = ep // NW
    n_ch = per_w // cb
    ident = NEG_INF if op == "max" else 0.0

    def opfn(a, b):
        return jnp.maximum(a, b) if op == "max" else a + b

    def body(d_h, v_h, o_h, dv, vv, tab, scf, sci):
        w = _wid()
        base = w * per_w

        def init(i, _):
            tab[pl.ds(i * LN, LN)] = jnp.full((LN,), ident, jnp.float32)
            return 0

        lax.fori_loop(0, nt * h // LN, init, 0)

        def chunk(ci, _):
            off = base + ci * cb
            pltpu.sync_copy(d_h.at[pl.ds(off, cb)], dv)
            if not use_ones:
                pltpu.sync_copy(v_h.at[pl.ds(off * h, cb * h)], vv)

            def vec(j, _):
                d16 = dv[pl.ds(j * LN, LN)]
                dsort, perm = plsc.sort_key_val(d16, _iota16())
                last = _last_mask(sci, dsort)
                for k in range(h):
                    if use_ones:
                        vs = jnp.full((LN,), 1.0, jnp.float32)
                    else:
                        vs = plsc.load_gather(vv, [j * (LN * h) + perm * h + k])
                    for sft in (1, 2, 4, 8):
                        kp = _shift_up(sci, dsort, sft, -1)
                        vp = _shift_up(scf, vs, sft, ident)
                        vs = jnp.where(kp == dsort, opfn(vs, vp), vs)
                    tidx = dsort * h + k
                    old = plsc.load_gather(tab, [tidx])
                    plsc.store_scatter(tab, [tidx], opfn(old, vs), mask=last)
                return 0

            lax.fori_loop(0, cb // LN, vec, 0)
            return 0

        lax.fori_loop(0, n_ch, chunk, 0)
        pltpu.sync_copy(tab, o_h.at[w])

    return pl.kernel(
        body,
        out_type=jax.ShapeDtypeStruct((NW, nt * h), jnp.float32),
        mesh=_mesh(),
        name="scs_%s%d%s" % (op, h, "o" if use_ones else ""),
        compiler_params=_SC_PARAMS,
        scratch_types=[pltpu.VMEM((cb,), jnp.int32),
                       pltpu.VMEM((cb * h,), jnp.float32),
                       pltpu.VMEM((nt * h,), jnp.float32),
                       pltpu.VMEM((LN,), jnp.float32),
                       pltpu.VMEM((LN,), jnp.int32)],
    )(dstp, vals)


def sc_map(dstp, vals, table, nt, h, opkind, cb=512):
    ep = dstp.shape[0]
    per_w = ep // NW
    n_ch = per_w // cb

    def body(d_h, v_h, t_h, o_h, dv, vv, ov, tab):
        base = _wid() * per_w
        pltpu.sync_copy(t_h, tab)

        def chunk(ci, _):
            off = base + ci * cb
            pltpu.sync_copy(d_h.at[pl.ds(off, cb)], dv)
            pltpu.sync_copy(v_h.at[pl.ds(off * h, cb * h)], vv)

            def vec(j, _):
                d16 = dv[pl.ds(j * LN, LN)]
                for k in range(h):
                    vidx = j * (LN * h) + _iota16() * h + k
                    v16 = plsc.load_gather(vv, [vidx])
                    t16 = plsc.load_gather(tab, [d16 * h + k])
                    if opkind == "exp_sub":
                        o16 = jnp.exp(v16 - t16)
                    else:
                        o16 = v16 / (t16 + 1e-16)
                    plsc.store_scatter(ov, [vidx], o16)
                return 0

            lax.fori_loop(0, cb // LN, vec, 0)
            pltpu.sync_copy(ov, o_h.at[pl.ds(off * h, cb * h)])
            return 0

        lax.fori_loop(0, n_ch, chunk, 0)

    return pl.kernel(
        body,
        out_type=jax.ShapeDtypeStruct((ep * h,), jnp.float32),
        mesh=_mesh(),
        name="scm_%s%d" % (opkind, h),
        compiler_params=_SC_PARAMS,
        scratch_types=[pltpu.VMEM((cb,), jnp.int32),
                       pltpu.VMEM((cb * h,), jnp.float32),
                       pltpu.VMEM((cb * h,), jnp.float32),
                       pltpu.VMEM((nt * h,), jnp.float32)],
    )(dstp, vals, table)


def sc_scatter_rows(wrows, dstp, nq, npass, cb, ks, zr=8):
    ep, fp = wrows.shape
    per_s = ep // NS
    n_ch = per_s // cb
    sb = cb // ks
    fs = nq // NS
    zrows = nq + LN
    nzch = (fs + LN + zr - 1) // zr

    def body(w_h, d_h, o_h, shared, rowv, dv, idx2, zbuf, sem):
        c = lax.axis_index("c")
        s = lax.axis_index("s")
        for r in range(zr):
            for f in range(fp // LN):
                zbuf[r, pl.ds(f * LN, LN)] = jnp.zeros((LN,), jnp.float32)

        for p in range(npass):
            q = c * npass + p
            qbase = q * nq

            def zero(i, _):
                lo = jnp.minimum(s * (zrows // NS) + i * zr, zrows - zr)
                pltpu.sync_copy(zbuf, shared.at[pl.ds(lo, zr)])
                return 0

            lax.fori_loop(0, nzch, zero, 0)
            plsc.subcore_barrier()

            def chunk(ci, _):
                off = s * per_s + ci * cb
                pltpu.sync_copy(d_h.at[pl.ds(off, cb)], dv)
                pltpu.sync_copy(w_h.at[pl.ds(off, cb)], rowv)

                for j in range(cb // LN):
                    d16 = dv[pl.ds(j * LN, LN)]
                    inq = (d16 >= qbase) & (d16 < qbase + nq)
                    li = jnp.where(inq, d16 - qbase, nq)
                    idx2[j * LN // sb, pl.ds(j * LN % sb, LN)] = li

                descs = []
                for j in range(ks):
                    descs.append(pltpu.async_copy(
                        rowv.at[pl.ds(j * sb, sb)],
                        shared.at[idx2.at[j]], sem, add=True))
                for d in descs:
                    d.wait()
                return 0

            lax.fori_loop(0, n_ch, chunk, 0)
            plsc.subcore_barrier()
            pltpu.sync_copy(shared.at[pl.ds(s * fs, fs)],
                            o_h.at[pl.ds(qbase + s * fs, fs)])
            plsc.subcore_barrier()

    return pl.kernel(
        body,
        out_type=jax.ShapeDtypeStruct((NC * npass * nq, fp), jnp.float32),
        mesh=_mesh(),
        name="scx%d" % fp,
        compiler_params=_SC_PARAMS,
        scratch_types=[pltpu.VMEM_SHARED((zrows, fp), jnp.float32),
                       pltpu.VMEM((cb, fp), jnp.float32),
                       pltpu.VMEM((cb,), jnp.int32),
                       pltpu.VMEM((ks, sb), jnp.int32),
                       pltpu.VMEM((zr, fp), jnp.float32),
                       pltpu.SemaphoreType.DMA],
    )(wrows, dstp)


def _pad_cols(x, kp):
    return jnp.pad(x, ((0, 0), (0, kp - x.shape[1])))


def _pad_rows(x, rp):
    return jnp.pad(x, ((0, rp - x.shape[0]), (0, 0)))


def _pad_w(w, b, fp):
    kp = _rup(w.shape[0], 16)
    wp = jnp.pad(w, ((0, kp - w.shape[0]), (0, fp - w.shape[1])))
    bp = jnp.pad(b, (0, fp - b.shape[0]))
    return wp, bp


def _prep_edges(src, dst, num_dst):
    e = src.shape[0]
    epad = _rup(e, 16384)
    srcp = jnp.concatenate(
        [src.astype(jnp.int32), jnp.zeros((epad - e,), jnp.int32)])
    dstp = jnp.concatenate(
        [dst.astype(jnp.int32), jnp.full((epad - e,), num_dst, jnp.int32)])
    return srcp, dstp


def _qcfg(fp):
    return {48: (1, 5632, 512, 4),
            64: (1, 25600, 256, 2),
            128: (3, 8704, 256, 2)}[fp]


def _deg(dstp, num_dst, nt):
    parts = sc_stats(dstp, dstp.astype(jnp.float32), nt, 1, "add",
                     use_ones=True)
    return tc_merge(parts, "add")[:num_dst]


def _gat_layer(xsrc_p, xdst_p, srcp, dstp, p, heads, cc, num_dst, nt, deg):
    fp = _rup(heads * cc, 16)
    wl, bl = _pad_w(p["Wl"], p["bl"], fp)
    wr, br = _pad_w(p["Wr"], p["br"], fp)
    np_src = xsrc_p.shape[0]
    np_dst = xdst_p.shape[0]
    xl = _pad_rows(tc_matmul(xsrc_p, wl)[:np_src] + bl, np_src)
    xr = tc_matmul(xdst_p, wr)[:np_dst] + br

    el, er = sc_gather2(xl, xr, srcp, dstp, cb=128, kf=(1 if fp == 128 else 2))
    amat = jnp.zeros((fp, heads), jnp.float32)
    for k in range(heads):
        amat = amat.at[k * cc:(k + 1) * cc, k].set(p["att"][k])
    alpha = tc_alpha(el, er, amat).reshape(-1)

    pmax = sc_stats(dstp, alpha, nt, heads, "max")
    amax = tc_merge(pmax, "max")
    expa = sc_map(dstp, alpha, amax, nt, heads, "exp_sub")
    psum = sc_stats(dstp, expa, nt, heads, "add")
    asum = tc_merge(psum, "add")
    alphan = sc_map(dstp, expa, asum, nt, heads, "div")

    wrows = tc_scale(el, alphan.reshape(-1, heads), heads)
    npass, nq, scb, sks = _qcfg(fp)
    agg = sc_scatter_rows(wrows, dstp, nq, npass, scb, sks)

    out = agg[:num_dst] / jnp.clip(deg, 1.0)[:, None]
    out = out[:, :heads * cc] + p["b"][None, :]
    return out


def _bn(x, p):
    return (x - p["mean"]) / jnp.sqrt(p["var"] + 1e-5) * p["gamma"] + p["beta"]


def _cnn_encode(x, p):
    n = x.shape[0]
    h = x
    for i in range(3):
        c = p["conv%d" % i]
        h = jax.lax.conv_general_dilated(
            h, c["w"], (1, 1), ((1, 1), (1, 1)),
            dimension_numbers=("NCHW", "OIHW", "NCHW"),
            feature_group_count=5) + c["b"][None, :, None, None]
        bnp = p["bn%d" % i]
        h = (h - bnp["mean"][None, :, None, None]) / jnp.sqrt(
            bnp["var"][None, :, None, None] + 1e-5) \
            * bnp["gamma"][None, :, None, None] + bnp["beta"][None, :, None, None]
        h = jax.nn.relu(h)
    h = jax.lax.reduce_window(h, -jnp.inf, jax.lax.max, (1, 1, 2, 2),
                              (1, 1, 2, 2), ((0, 0), (0, 0), (1, 1), (1, 1)))
    return h.reshape(n, -1)


def kernel(x_low, x_high, z_std, params, edge_index_low, edge_index_l2h,
           edge_index_high):
    np_low = N_LOW + 16
    np_high = N_HIGH + 16
    nt_low = _rup(N_LOW + 16, 2048)
    nt_high = _rup(N_HIGH + 16, 2048)

    s_l, d_l = _prep_edges(edge_index_low[0], edge_index_low[1], N_LOW)
    s_m, d_m = _prep_edges(edge_index_l2h[0], edge_index_l2h[1], N_HIGH)
    loop = jnp.arange(N_HIGH, dtype=edge_index_high.dtype)
    s_h, d_h = _prep_edges(jnp.concatenate([edge_index_high[0], loop]),
                           jnp.concatenate([edge_index_high[1], loop]), N_HIGH)
    deg_l = _deg(d_l, N_LOW, nt_low)
    deg_m = _deg(d_m, N_HIGH, nt_high)
    deg_h = _deg(d_h, N_HIGH, nt_high)

    h = _cnn_encode(x_low, params["cnn"])
    hp = _pad_rows(_pad_cols(h, 48), np_low)
    for p in params["gl"]:
        out = _gat_layer(hp, hp, s_l, d_l, p, 1, 45, N_LOW, nt_low, deg_l)
        out = jax.nn.relu(out)
        hp = _pad_rows(_pad_cols(out, 48), np_low)

    xh_p = _pad_rows(_pad_cols(x_high, 16), np_high)
    h2 = _gat_layer(hp, xh_p, s_m, d_m, params["down"], 1, 64, N_HIGH,
                    nt_high, deg_m)

    x = jnp.concatenate([z_std, h2], axis=-1)
    x = _bn(x, params["hbn0"])
    xp = _pad_rows(_pad_cols(x, 80), np_high)
    hcfg = [(2, 64), (2, 64), (2, 64), (2, 64), (1, 64)]
    for i, (hh, cc) in enumerate(hcfg):
        out = _gat_layer(xp, xp, s_h, d_h, params["hg"][i], hh, cc, N_HIGH,
                         nt_high, deg_h)
        if i < 4:
            out = jax.nn.relu(_bn(out, params["hbn"][i]))
        else:
            out = jax.nn.relu(out)
        xp = _pad_rows(out, np_high)

    pr = params["pred"]
    y = jax.nn.relu(tc_matmul(xp[:N_HIGH], pr["W1"]) + pr["b1"])
    y = jax.nn.relu(tc_matmul(y, pr["W2"]) + pr["b2"])
    return tc_matmul(y, pr["W3"]) + pr["b3"]


# speedup vs baseline: 1.1356x; 1.1356x over previous
import functools
import jax
import jax.numpy as jnp
import numpy as np
from jax import lax
from jax.experimental import pallas as pl
from jax.experimental.pallas import tpu as pltpu
from jax.experimental.pallas import tpu_sc as plsc

N_LOW = 10000
N_HIGH = 50000

NC, NS, LN = 2, 16, 16
NW = NC * NS
NEG_INF = float("-inf")


def _rup(x, m):
    return (x + m - 1) // m * m


def _mesh():
    return plsc.VectorSubcoreMesh(core_axis_name="c", subcore_axis_name="s")


_SC_PARAMS = pltpu.CompilerParams(use_tc_tiling_on_sc=False,
                                  needs_layout_passes=False)


def _wid():
    return lax.axis_index("s") * NC + lax.axis_index("c")


def _iota16():
    return jnp.arange(LN, dtype=jnp.int32)


def _take16(scr, x, idx):
    scr[...] = x
    return plsc.load_gather(scr, [idx])


def _shift_up(scr, x, s, ident):
    io = _iota16()
    g = _take16(scr, x, jnp.maximum(io - s, 0))
    return jnp.where(io >= s, g, ident)


def _last_mask(scr_i, d):
    io = _iota16()
    nxt = _take16(scr_i, d, jnp.minimum(io + 1, LN - 1))
    return (io == LN - 1) | (d != nxt)


def _mm_body(x_ref, w_ref, o_ref):
    o_ref[...] = jnp.dot(x_ref[...], w_ref[...],
                         preferred_element_type=jnp.float32)


def tc_matmul(x, w, bm=1024):
    m, k = x.shape
    n = w.shape[1]
    mp = _rup(m, bm)
    if mp != m:
        x = jnp.pad(x, ((0, mp - m), (0, 0)))
    out = pl.pallas_call(
        _mm_body,
        grid=(mp // bm,),
        in_specs=[pl.BlockSpec((bm, k), lambda i: (i, 0)),
                  pl.BlockSpec((k, n), lambda i: (0, 0))],
        out_specs=pl.BlockSpec((bm, n), lambda i: (i, 0)),
        out_shape=jax.ShapeDtypeStruct((mp, n), jnp.float32),
    )(x, w)
    return out[:m]


def _alpha_body(el_ref, er_ref, a_ref, o_ref):
    x = el_ref[...] + er_ref[...]
    g = jnp.maximum(x, 0.2 * x)
    o_ref[...] = jnp.dot(g, a_ref[...], preferred_element_type=jnp.float32)


def tc_alpha(el, er, amat, be=2048):
    ep, fp = el.shape
    h = amat.shape[1]
    return pl.pallas_call(
        _alpha_body,
        grid=(ep // be,),
        in_specs=[pl.BlockSpec((be, fp), lambda i: (i, 0)),
                  pl.BlockSpec((be, fp), lambda i: (i, 0)),
                  pl.BlockSpec((fp, h), lambda i: (0, 0))],
        out_specs=pl.BlockSpec((be, h), lambda i: (i, 0)),
        out_shape=jax.ShapeDtypeStruct((ep, h), jnp.float32),
    )(el, er, amat)


def tc_scale(el, al, heads, be=2048):
    ep, fp = el.shape
    cc = fp // heads

    def body(el_ref, al_ref, o_ref):
        parts = [el_ref[:, k * cc:(k + 1) * cc] * al_ref[:, k:k + 1]
                 for k in range(heads)]
        o_ref[...] = jnp.concatenate(parts, axis=1) if heads > 1 else parts[0]

    return pl.pallas_call(
        body,
        grid=(ep // be,),
        in_specs=[pl.BlockSpec((be, fp), lambda i: (i, 0)),
                  pl.BlockSpec((be, heads), lambda i: (i, 0))],
        out_specs=pl.BlockSpec((be, fp), lambda i: (i, 0)),
        out_shape=jax.ShapeDtypeStruct((ep, fp), jnp.float32),
    )(el, al)


def tc_merge(partials, op, bm=1024):
    nw, m = partials.shape

    def body(p_ref, o_ref):
        x = p_ref[...]
        if op == "max":
            r = jnp.max(x, axis=0)
            r = jnp.where(jnp.isfinite(r), r, 0.0)
        else:
            r = jnp.sum(x, axis=0)
        o_ref[...] = r[None, :]

    out = pl.pallas_call(
        body,
        grid=(m // bm,),
        in_specs=[pl.BlockSpec((nw, bm), lambda i: (0, i))],
        out_specs=pl.BlockSpec((1, bm), lambda i: (0, i)),
        out_shape=jax.ShapeDtypeStruct((1, m), jnp.float32),
    )(partials)
    return out[0]


def sc_gather2(tabl, tabr, idxl, idxr, cb=128):
    ep = idxl.shape[0]
    fp = tabl.shape[1]
    per_w = ep // NW
    n_ch = per_w // cb

    def body(tl, tr, il, ir, ol, outr, ixl, ixr, rl0, rl1, rr0, rr1,
             gsem, osem0, osem1):
        base = _wid() * per_w
        rls, rrs, osems = (rl0, rl1), (rr0, rr1), (osem0, osem1)

        def chunk(i, _):
            for p in (0, 1):
                off = base + (2 * i + p) * cb

                @pl.when(i > 0)
                def _():
                    pltpu.make_async_copy(
                        rls[p], ol.at[pl.ds(off, cb)], osems[p]).wait()
                    pltpu.make_async_copy(
                        rrs[p], outr.at[pl.ds(off, cb)], osems[p]).wait()

                pltpu.sync_copy(il.at[pl.ds(off, cb)], ixl.at[p])
                pltpu.sync_copy(ir.at[pl.ds(off, cb)], ixr.at[p])
                pltpu.async_copy(tl.at[ixl.at[p]], rls[p], gsem)
                pltpu.async_copy(tr.at[ixr.at[p]], rrs[p], gsem)
            for p in (0, 1):
                off = base + (2 * i + p) * cb
                pltpu.make_async_copy(tl.at[ixl.at[p]], rls[p], gsem).wait()
                pltpu.make_async_copy(tr.at[ixr.at[p]], rrs[p], gsem).wait()
                pltpu.async_copy(rls[p], ol.at[pl.ds(off, cb)], osems[p])
                pltpu.async_copy(rrs[p], outr.at[pl.ds(off, cb)], osems[p])
            return 0

        lax.fori_loop(0, n_ch // 2, chunk, 0)
        for p in (0, 1):
            pltpu.make_async_copy(rls[p], ol.at[pl.ds(base, cb)],
                                  osems[p]).wait()
            pltpu.make_async_copy(rrs[p], outr.at[pl.ds(base, cb)],
                                  osems[p]).wait()

    shp = jax.ShapeDtypeStruct((ep, fp), jnp.float32)
    return pl.kernel(
        body,
        out_type=(shp, shp),
        mesh=_mesh(),
        name="scg%d" % fp,
        compiler_params=_SC_PARAMS,
        scratch_types=[pltpu.VMEM((2, cb), jnp.int32),
                       pltpu.VMEM((2, cb), jnp.int32),
                       pltpu.VMEM((cb, fp), jnp.float32),
                       pltpu.VMEM((cb, fp), jnp.float32),
                       pltpu.VMEM((cb, fp), jnp.float32),
                       pltpu.VMEM((cb, fp), jnp.float32),
                       pltpu.SemaphoreType.DMA,
                       pltpu.SemaphoreType.DMA,
                       pltpu.SemaphoreType.DMA],
    )(tabl, tabr, idxl, idxr)


def sc_stats(dstp, vals, nt, h, op, use_ones=False, cb=512):
    ep = dstp.shape[0]
    per_w = ep // NW
    n_ch = per_w // cb
    ident = NEG_INF if op == "max" else 0.0

    def opfn(a, b):
        return jnp.maximum(a, b) if op == "max" else a + b

    def body(d_h, v_h, o_h, dv, vv, tab, scf, sci):
        w = _wid()
        base = w * per_w

        def init(i, _):
            tab[pl.ds(i * LN, LN)] = jnp.full((LN,), ident, jnp.float32)
            return 0

        lax.fori_loop(0, nt * h // LN, init, 0)

        def chunk(ci, _):
            off = base + ci * cb
            pltpu.sync_copy(d_h.at[pl.ds(off, cb)], dv)
            if not use_ones:
                pltpu.sync_copy(v_h.at[pl.ds(off * h, cb * h)], vv)

            def vec(j, _):
                d16 = dv[pl.ds(j * LN, LN)]
                dsort, perm = plsc.sort_key_val(d16, _iota16())
                last = _last_mask(sci, dsort)
                for k in range(h):
                    if use_ones:
                        vs = jnp.full((LN,), 1.0, jnp.float32)
                    else:
                        vs = plsc.load_gather(vv, [j * (LN * h) + perm * h + k])
                    for sft in (1, 2, 4, 8):
                        kp = _shift_up(sci, dsort, sft, -1)
                        vp = _shift_up(scf, vs, sft, ident)
                        vs = jnp.where(kp == dsort, opfn(vs, vp), vs)
                    tidx = dsort * h + k
                    old = plsc.load_gather(tab, [tidx])
                    plsc.store_scatter(tab, [tidx], opfn(old, vs), mask=last)
                return 0

            lax.fori_loop(0, cb // LN, vec, 0)
            return 0

        lax.fori_loop(0, n_ch, chunk, 0)
        pltpu.sync_copy(tab, o_h.at[w])

    return pl.kernel(
        body,
        out_type=jax.ShapeDtypeStruct((NW, nt * h), jnp.float32),
        mesh=_mesh(),
        name="scs_%s%d%s" % (op, h, "o" if use_ones else ""),
        compiler_params=_SC_PARAMS,
        scratch_types=[pltpu.VMEM((cb,), jnp.int32),
                       pltpu.VMEM((cb * h,), jnp.float32),
                       pltpu.VMEM((nt * h,), jnp.float32),
                       pltpu.VMEM((LN,), jnp.float32),
                       pltpu.VMEM((LN,), jnp.int32)],
    )(dstp, vals)


def sc_map(dstp, vals, table, nt, h, opkind, cb=512):
    ep = dstp.shape[0]
    per_w = ep // NW
    n_ch = per_w // cb

    def body(d_h, v_h, t_h, o_h, dv, vv, ov, tab):
        base = _wid() * per_w
        pltpu.sync_copy(t_h, tab)

        def chunk(ci, _):
            off = base + ci * cb
            pltpu.sync_copy(d_h.at[pl.ds(off, cb)], dv)
            pltpu.sync_copy(v_h.at[pl.ds(off * h, cb * h)], vv)

            def vec(j, _):
                d16 = dv[pl.ds(j * LN, LN)]
                for k in range(h):
                    vidx = j * (LN * h) + _iota16() * h + k
                    v16 = plsc.load_gather(vv, [vidx])
                    t16 = plsc.load_gather(tab, [d16 * h + k])
                    if opkind == "exp_sub":
                        o16 = jnp.exp(v16 - t16)
                    else:
                        o16 = v16 / (t16 + 1e-16)
                    plsc.store_scatter(ov, [vidx], o16)
                return 0

            lax.fori_loop(0, cb // LN, vec, 0)
            pltpu.sync_copy(ov, o_h.at[pl.ds(off * h, cb * h)])
            return 0

        lax.fori_loop(0, n_ch, chunk, 0)

    return pl.kernel(
        body,
        out_type=jax.ShapeDtypeStruct((ep * h,), jnp.float32),
        mesh=_mesh(),
        name="scm_%s%d" % (opkind, h),
        compiler_params=_SC_PARAMS,
        scratch_types=[pltpu.VMEM((cb,), jnp.int32),
                       pltpu.VMEM((cb * h,), jnp.float32),
                       pltpu.VMEM((cb * h,), jnp.float32),
                       pltpu.VMEM((nt * h,), jnp.float32)],
    )(dstp, vals, table)


def sc_scatter_rows(wrows, dstp, nq, npass, cb, ks, zr=8):
    ep, fp = wrows.shape
    per_s = ep // NS
    n_ch = per_s // cb
    sb = cb // ks
    fs = nq // NS
    zrows = nq + LN
    nzch = (fs + LN + zr - 1) // zr

    def body(w_h, d_h, o_h, shared, rowv, dv, idx2, zbuf, sem):
        c = lax.axis_index("c")
        s = lax.axis_index("s")
        for r in range(zr):
            for f in range(fp // LN):
                zbuf[r, pl.ds(f * LN, LN)] = jnp.zeros((LN,), jnp.float32)

        for p in range(npass):
            q = c * npass + p
            qbase = q * nq

            def zero(i, _):
                lo = jnp.minimum(s * (zrows // NS) + i * zr, zrows - zr)
                pltpu.sync_copy(zbuf, shared.at[pl.ds(lo, zr)])
                return 0

            lax.fori_loop(0, nzch, zero, 0)
            plsc.subcore_barrier()

            def chunk(ci, _):
                off = s * per_s + ci * cb
                pltpu.sync_copy(d_h.at[pl.ds(off, cb)], dv)
                pltpu.sync_copy(w_h.at[pl.ds(off, cb)], rowv)

                for j in range(cb // LN):
                    d16 = dv[pl.ds(j * LN, LN)]
                    inq = (d16 >= qbase) & (d16 < qbase + nq)
                    li = jnp.where(inq, d16 - qbase, nq)
                    idx2[j * LN // sb, pl.ds(j * LN % sb, LN)] = li

                descs = []
                for j in range(ks):
                    descs.append(pltpu.async_copy(
                        rowv.at[pl.ds(j * sb, sb)],
                        shared.at[idx2.at[j]], sem, add=True))
                for d in descs:
                    d.wait()
                return 0

            lax.fori_loop(0, n_ch, chunk, 0)
            plsc.subcore_barrier()
            pltpu.sync_copy(shared.at[pl.ds(s * fs, fs)],
                            o_h.at[pl.ds(qbase + s * fs, fs)])
            plsc.subcore_barrier()

    return pl.kernel(
        body,
        out_type=jax.ShapeDtypeStruct((NC * npass * nq, fp), jnp.float32),
        mesh=_mesh(),
        name="scx%d" % fp,
        compiler_params=_SC_PARAMS,
        scratch_types=[pltpu.VMEM_SHARED((zrows, fp), jnp.float32),
                       pltpu.VMEM((cb, fp), jnp.float32),
                       pltpu.VMEM((cb,), jnp.int32),
                       pltpu.VMEM((ks, sb), jnp.int32),
                       pltpu.VMEM((zr, fp), jnp.float32),
                       pltpu.SemaphoreType.DMA],
    )(wrows, dstp)


def _pad_cols(x, kp):
    return jnp.pad(x, ((0, 0), (0, kp - x.shape[1])))


def _pad_rows(x, rp):
    return jnp.pad(x, ((0, rp - x.shape[0]), (0, 0)))


def _pad_w(w, b, fp):
    kp = _rup(w.shape[0], 16)
    wp = jnp.pad(w, ((0, kp - w.shape[0]), (0, fp - w.shape[1])))
    bp = jnp.pad(b, (0, fp - b.shape[0]))
    return wp, bp


def _prep_edges(src, dst, num_dst):
    e = src.shape[0]
    epad = _rup(e, 16384)
    srcp = jnp.concatenate(
        [src.astype(jnp.int32), jnp.zeros((epad - e,), jnp.int32)])
    dstp = jnp.concatenate(
        [dst.astype(jnp.int32), jnp.full((epad - e,), num_dst, jnp.int32)])
    return srcp, dstp


def _qcfg(fp):
    return {48: (1, 5632, 512, 4),
            64: (1, 25600, 256, 2),
            128: (2, 12800, 128, 1)}[fp]


def _deg(dstp, num_dst, nt):
    parts = sc_stats(dstp, dstp.astype(jnp.float32), nt, 1, "add",
                     use_ones=True)
    return tc_merge(parts, "add")[:num_dst]


def _gat_layer(xsrc_p, xdst_p, srcp, dstp, p, heads, cc, num_dst, nt, deg):
    fp = _rup(heads * cc, 16)
    wl, bl = _pad_w(p["Wl"], p["bl"], fp)
    wr, br = _pad_w(p["Wr"], p["br"], fp)
    np_src = xsrc_p.shape[0]
    np_dst = xdst_p.shape[0]
    xl = _pad_rows(tc_matmul(xsrc_p, wl)[:np_src] + bl, np_src)
    xr = tc_matmul(xdst_p, wr)[:np_dst] + br

    el, er = sc_gather2(xl, xr, srcp, dstp)
    amat = jnp.zeros((fp, heads), jnp.float32)
    for k in range(heads):
        amat = amat.at[k * cc:(k + 1) * cc, k].set(p["att"][k])
    alpha = tc_alpha(el, er, amat).reshape(-1)

    pmax = sc_stats(dstp, alpha, nt, heads, "max")
    amax = tc_merge(pmax, "max")
    expa = sc_map(dstp, alpha, amax, nt, heads, "exp_sub")
    psum = sc_stats(dstp, expa, nt, heads, "add")
    asum = tc_merge(psum, "add")
    alphan = sc_map(dstp, expa, asum, nt, heads, "div")

    wrows = tc_scale(el, alphan.reshape(-1, heads), heads)
    npass, nq, scb, sks = _qcfg(fp)
    agg = sc_scatter_rows(wrows, dstp, nq, npass, scb, sks)

    out = agg[:num_dst] / jnp.clip(deg, 1.0)[:, None]
    out = out[:, :heads * cc] + p["b"][None, :]
    return out


def _bn(x, p):
    return (x - p["mean"]) / jnp.sqrt(p["var"] + 1e-5) * p["gamma"] + p["beta"]


def _cnn_encode(x, p):
    n = x.shape[0]
    h = x
    for i in range(3):
        c = p["conv%d" % i]
        h = jax.lax.conv_general_dilated(
            h, c["w"], (1, 1), ((1, 1), (1, 1)),
            dimension_numbers=("NCHW", "OIHW", "NCHW"),
            feature_group_count=5) + c["b"][None, :, None, None]
        bnp = p["bn%d" % i]
        h = (h - bnp["mean"][None, :, None, None]) / jnp.sqrt(
            bnp["var"][None, :, None, None] + 1e-5) \
            * bnp["gamma"][None, :, None, None] + bnp["beta"][None, :, None, None]
        h = jax.nn.relu(h)
    h = jax.lax.reduce_window(h, -jnp.inf, jax.lax.max, (1, 1, 2, 2),
                              (1, 1, 2, 2), ((0, 0), (0, 0), (1, 1), (1, 1)))
    return h.reshape(n, -1)


def kernel(x_low, x_high, z_std, params, edge_index_low, edge_index_l2h,
           edge_index_high):
    np_low = N_LOW + 16
    np_high = N_HIGH + 16
    nt_low = _rup(N_LOW + 16, 2048)
    nt_high = _rup(N_HIGH + 16, 2048)

    s_l, d_l = _prep_edges(edge_index_low[0], edge_index_low[1], N_LOW)
    s_m, d_m = _prep_edges(edge_index_l2h[0], edge_index_l2h[1], N_HIGH)
    loop = jnp.arange(N_HIGH, dtype=edge_index_high.dtype)
    s_h, d_h = _prep_edges(jnp.concatenate([edge_index_high[0], loop]),
                           jnp.concatenate([edge_index_high[1], loop]), N_HIGH)
    deg_l = _deg(d_l, N_LOW, nt_low)
    deg_m = _deg(d_m, N_HIGH, nt_high)
    deg_h = _deg(d_h, N_HIGH, nt_high)

    h = _cnn_encode(x_low, params["cnn"])
    hp = _pad_rows(_pad_cols(h, 48), np_low)
    for p in params["gl"]:
        out = _gat_layer(hp, hp, s_l, d_l, p, 1, 45, N_LOW, nt_low, deg_l)
        out = jax.nn.relu(out)
        hp = _pad_rows(_pad_cols(out, 48), np_low)

    xh_p = _pad_rows(_pad_cols(x_high, 16), np_high)
    h2 = _gat_layer(hp, xh_p, s_m, d_m, params["down"], 1, 64, N_HIGH,
                    nt_high, deg_m)

    x = jnp.concatenate([z_std, h2], axis=-1)
    x = _bn(x, params["hbn0"])
    xp = _pad_rows(_pad_cols(x, 80), np_high)
    hcfg = [(2, 64), (2, 64), (2, 64), (2, 64), (1, 64)]
    for i, (hh, cc) in enumerate(hcfg):
        out = _gat_layer(xp, xp, s_h, d_h, params["hg"][i], hh, cc, N_HIGH,
                         nt_high, deg_h)
        if i < 4:
            out = jax.nn.relu(_bn(out, params["hbn"][i]))
        else:
            out = jax.nn.relu(out)
        xp = _pad_rows(out, np_high)

    pr = params["pred"]
    y = jax.nn.relu(tc_matmul(xp[:N_HIGH], pr["W1"]) + pr["b1"])
    y = jax.nn.relu(tc_matmul(y, pr["W2"]) + pr["b2"])
    return tc_matmul(y, pr["W3"]) + pr["b3"]
